# baseline (speedup 1.0000x reference)
import jax
import jax.numpy as jnp
from jax.experimental import pallas as pl
from jax.experimental.pallas import tpu as pltpu

DICT_LEN = 9
NUM_LS = 8
DIM = 1024
SEQ = 2048
BZ = 16
BLK_S = 512


def _mapper_kernel(idx_ref, x_ref, w_ref, b_ref, o_ref):
    xj = x_ref[:, 0, 0, :].astype(jnp.bfloat16)
    yj = jax.lax.dot_general(
        xj, w_ref[0],
        dimension_numbers=(((1,), (0,)), ((), ())),
        preferred_element_type=jnp.float32,
    )
    o_ref[:, 0, 0, :] = yj + b_ref[0, 0]


def kernel(x, lang_ids, W, b):
    idx = jnp.clip(DICT_LEN - 2 - lang_ids, 0, NUM_LS - 1).astype(jnp.int32)
    Wt = jnp.swapaxes(W, 1, 2).astype(jnp.bfloat16)
    x4 = x.reshape(SEQ, BZ, 1, DIM)
    b3 = b.reshape(NUM_LS, 1, DIM)
    grid = (BZ, SEQ // BLK_S)
    out = pl.pallas_call(
        _mapper_kernel,
        grid_spec=pltpu.PrefetchScalarGridSpec(
            num_scalar_prefetch=1,
            grid=grid,
            in_specs=[
                pl.BlockSpec((BLK_S, 1, 1, DIM), lambda bi, si, idx_ref: (si, bi, 0, 0)),
                pl.BlockSpec((1, DIM, DIM), lambda bi, si, idx_ref: (idx_ref[bi], 0, 0)),
                pl.BlockSpec((1, 1, DIM), lambda bi, si, idx_ref: (idx_ref[bi], 0, 0)),
            ],
            out_specs=pl.BlockSpec((BLK_S, 1, 1, DIM), lambda bi, si, idx_ref: (si, bi, 0, 0)),
        ),
        out_shape=jax.ShapeDtypeStruct((SEQ, BZ, 1, DIM), jnp.float32),
    )(idx, x4, Wt, b3)
    return out.reshape(SEQ, BZ, DIM)


# speedup vs baseline: 1.5233x; 1.5233x over previous
import jax
import jax.numpy as jnp
from jax.experimental import pallas as pl
from jax.experimental.pallas import tpu as pltpu

DICT_LEN = 9
NUM_LS = 8
DIM = 1024
SEQ = 2048
BZ = 16
BLK_S = 128


def _mapper_kernel(idx_ref, x_ref, w_ref, b_ref, o_ref, xcol, sem):
    def copy(j, slot):
        return pltpu.make_async_copy(
            x_ref.at[:, j, :], xcol.at[slot], sem.at[slot]
        )

    copy(0, 0).start()
    for j in range(BZ):
        slot = j % 2
        if j + 1 < BZ:
            copy(j + 1, 1 - slot).start()
        copy(j, slot).wait()
        e = idx_ref[j]
        xj = xcol[slot].astype(jnp.bfloat16)
        yj = jax.lax.dot_general(
            xj, w_ref[e],
            dimension_numbers=(((1,), (0,)), ((), ())),
            preferred_element_type=jnp.float32,
        )
        o_ref[:, j, :] = yj + b_ref[e]


def kernel(x, lang_ids, W, b):
    idx = jnp.clip(DICT_LEN - 2 - lang_ids, 0, NUM_LS - 1).astype(jnp.int32)
    Wt = jnp.swapaxes(W, 1, 2).astype(jnp.bfloat16)
    grid = (SEQ // BLK_S,)
    out = pl.pallas_call(
        _mapper_kernel,
        grid_spec=pltpu.PrefetchScalarGridSpec(
            num_scalar_prefetch=1,
            grid=grid,
            in_specs=[
                pl.BlockSpec((BLK_S, BZ, DIM), lambda s, idx_ref: (s, 0, 0)),
                pl.BlockSpec((NUM_LS, DIM, DIM), lambda s, idx_ref: (0, 0, 0)),
                pl.BlockSpec((NUM_LS, DIM), lambda s, idx_ref: (0, 0)),
            ],
            out_specs=pl.BlockSpec((BLK_S, BZ, DIM), lambda s, idx_ref: (s, 0, 0)),
            scratch_shapes=[
                pltpu.VMEM((2, BLK_S, DIM), jnp.float32),
                pltpu.SemaphoreType.DMA((2,)),
            ],
        ),
        out_shape=jax.ShapeDtypeStruct((SEQ, BZ, DIM), jnp.float32),
    )(idx, x, Wt, b)
    return out
